# baseline (speedup 1.0000x reference)
import jax
import jax.numpy as jnp
from jax import lax
from jax.experimental import pallas as pl
from jax.experimental import layout as jex_layout
from jax.experimental.pallas import tpu as pltpu
from jax.experimental.pallas import tpu_sc as plsc

VOCAB = 1000000
SEQ_LEN = 200
EMBED_DIM = 64
BATCH = 4096

NUM_CORES = 2
NUM_SUBCORES = 16
NUM_WORKERS = NUM_CORES * NUM_SUBCORES
SEQ_PER_WORKER = BATCH // NUM_WORKERS
LANES = 16
VREGS_PER_ROW = EMBED_DIM // LANES

NUM_CHUNKS = SEQ_PER_WORKER
NBUF = 7
LOOKAHEAD = NBUF - 1

GATHER_SPLITS = ((0, 128), (128, 72))

EMBED_SCALE = 8.0


def _sc_kernel_body(inputs_hbm, tok_hbm, pos_hbm, out_hbm, *scratch):
    idx = scratch[0:NBUF]
    rows = scratch[NBUF:2 * NBUF]
    pos_v = scratch[2 * NBUF]
    sem_g = scratch[2 * NBUF + 1:3 * NBUF + 1]
    sem_o = scratch[3 * NBUF + 1:4 * NBUF + 1]
    sem_i = scratch[4 * NBUF + 1:5 * NBUF + 1]

    wid = lax.axis_index("s") * NUM_CORES + lax.axis_index("c")
    base_seq = wid * SEQ_PER_WORKER

    pltpu.sync_copy(pos_hbm, pos_v)

    def idx_io(g, b, start):
        desc = pltpu.make_async_copy(
            inputs_hbm.at[pl.ds(base_seq + g, 1)], idx[b], sem_i[b])
        desc.start() if start else desc.wait()

    def gather_io(b, start):
        for (off, n) in GATHER_SPLITS:
            desc = pltpu.make_async_copy(
                tok_hbm.at[idx[b].at[0, pl.ds(off, n)]],
                rows[b].at[pl.ds(off, n)],
                sem_g[b])
            desc.start() if start else desc.wait()

    def out_io(g, b, start):
        desc = pltpu.make_async_copy(
            rows[b],
            out_hbm.at[base_seq + g, :, pl.ds(0, EMBED_DIM)],
            sem_o[b])
        desc.start() if start else desc.wait()

    def compute(b):
        def row_body(t, c2):
            for c in range(VREGS_PER_ROW):
                sl = pl.ds(c * LANES, LANES)
                rows[b][t, sl] = rows[b][t, sl] * EMBED_SCALE + pos_v[t, sl]
            return c2

        lax.fori_loop(0, SEQ_LEN, row_body, 0, unroll=4)

    for j in range(NBUF):
        idx_io(j, j, True)
    for j in range(LOOKAHEAD):
        idx_io(j, j, False)
        gather_io(j, True)

    def outer_body(gg, carry):
        for b in range(NBUF):
            g = gg * NBUF + b
            nb = (b + LOOKAHEAD) % NBUF

            @pl.when(g >= 1)
            def _():
                out_io(g - 1, nb, False)

            @pl.when(g + LOOKAHEAD < NUM_CHUNKS)
            def _():
                idx_io(g + LOOKAHEAD, nb, False)
                gather_io(nb, True)

            gather_io(b, False)

            @pl.when(g + NBUF < NUM_CHUNKS)
            def _():
                idx_io(g + NBUF, b, True)

            compute(b)
            out_io(g, b, True)
        return carry

    assert NUM_CHUNKS % NBUF == 0 or True
    n_full = NUM_CHUNKS // NBUF
    lax.fori_loop(0, n_full, outer_body, 0)
    for g in range(n_full * NBUF, NUM_CHUNKS):
        b = g % NBUF
        nb = (b + LOOKAHEAD) % NBUF
        out_io(g - 1, nb, False)
        if g + LOOKAHEAD < NUM_CHUNKS:
            idx_io(g + LOOKAHEAD, nb, False)
            gather_io(nb, True)
        gather_io(b, False)
        compute(b)
        out_io(g, b, True)

    out_io(NUM_CHUNKS - 1, (NUM_CHUNKS - 1) % NBUF, False)


@jax.jit
def _embed(inputs, tok_table, pos_table):
    mesh = plsc.VectorSubcoreMesh(core_axis_name="c", subcore_axis_name="s")
    fn = pl.kernel(
        _sc_kernel_body,
        out_type=jax.ShapeDtypeStruct((BATCH, SEQ_LEN, 2 * EMBED_DIM),
                                      jnp.float32),
        mesh=mesh,
        scratch_types=(
            [pltpu.VMEM((1, SEQ_LEN), jnp.int32)] * NBUF
            + [pltpu.VMEM((SEQ_LEN, EMBED_DIM), jnp.float32)] * NBUF
            + [pltpu.VMEM((SEQ_LEN, EMBED_DIM), jnp.float32)]
            + [pltpu.SemaphoreType.DMA] * (3 * NBUF)
        ),
        compiler_params=pltpu.CompilerParams(use_tc_tiling_on_sc=False),
    )
    return fn(inputs, tok_table, pos_table)


def kernel(inputs, tok_table, pos_table):
    return _embed(inputs, tok_table, pos_table)[:, :, :EMBED_DIM]


# speedup vs baseline: 1.1866x; 1.1866x over previous
import jax
import jax.numpy as jnp
from jax import lax
from jax.experimental import pallas as pl
from jax.experimental.pallas import tpu as pltpu
from jax.experimental.pallas import tpu_sc as plsc

VOCAB = 1000000
SEQ_LEN = 200
EMBED_DIM = 64
BATCH = 4096

NUM_CORES = 2
NUM_SUBCORES = 16
NUM_WORKERS = NUM_CORES * NUM_SUBCORES
SEQ_PER_WORKER = BATCH // NUM_WORKERS
LANES = 16
VREGS_PER_ROW = EMBED_DIM // LANES

SEQ_PER_CHUNK = 2
ROWS_PER_CHUNK = SEQ_PER_CHUNK * SEQ_LEN
NUM_CHUNKS = SEQ_PER_WORKER // SEQ_PER_CHUNK
NBUF = 4

GATHER_SPLITS = ((0, 128), (128, 72))

EMBED_SCALE = 8.0


def _sc_kernel_body(inputs_hbm, tok_hbm, pos_hbm, out_hbm,
                    idx0, idx1, idx2, idx3,
                    rows0, rows1, rows2, rows3, pos_v,
                    sg0, sg1, sg2, sg3, so0, so1, so2, so3,
                    si0, si1, si2, si3):
    wid = lax.axis_index("s") * NUM_CORES + lax.axis_index("c")
    base_seq = wid * SEQ_PER_WORKER

    idx = (idx0, idx1, idx2, idx3)
    rows = (rows0, rows1, rows2, rows3)
    sem_g = (sg0, sg1, sg2, sg3)
    sem_o = (so0, so1, so2, so3)
    sem_i = (si0, si1, si2, si3)

    pltpu.sync_copy(pos_hbm, pos_v)

    def idx_io(g, b, start):
        desc = pltpu.make_async_copy(
            inputs_hbm.at[pl.ds(base_seq + g * SEQ_PER_CHUNK, SEQ_PER_CHUNK)],
            idx[b], sem_i[b])
        desc.start() if start else desc.wait()

    def gather_io(b, start):
        for s in range(SEQ_PER_CHUNK):
            for (off, n) in GATHER_SPLITS:
                desc = pltpu.make_async_copy(
                    tok_hbm.at[idx[b].at[s, pl.ds(off, n)]],
                    rows[b].at[pl.ds(s * SEQ_LEN + off, n)],
                    sem_g[b])
                desc.start() if start else desc.wait()

    def out_io(g, b, start):
        for s in range(SEQ_PER_CHUNK):
            desc = pltpu.make_async_copy(
                rows[b].at[pl.ds(s * SEQ_LEN, SEQ_LEN)],
                out_hbm.at[base_seq + g * SEQ_PER_CHUNK + s, :,
                           pl.ds(0, EMBED_DIM)],
                sem_o[b])
            desc.start() if start else desc.wait()

    def compute(b):
        def row_body(t, c2):
            for c in range(VREGS_PER_ROW):
                sl = pl.ds(c * LANES, LANES)
                p = pos_v[t, sl]
                for s in range(SEQ_PER_CHUNK):
                    r = s * SEQ_LEN + t
                    rows[b][r, sl] = rows[b][r, sl] * EMBED_SCALE + p
            return c2

        lax.fori_loop(0, SEQ_LEN, row_body, 0, unroll=2)

    for j in range(NBUF):
        idx_io(j, j, True)
    for j in range(NBUF - 1):
        idx_io(j, j, False)
        gather_io(j, True)

    def outer_body(gg, carry):
        for b in range(NBUF):
            g = gg * NBUF + b
            nb = (b + 3) % NBUF

            @pl.when(g >= 1)
            def _():
                out_io(g - 1, nb, False)

            @pl.when(g + 3 < NUM_CHUNKS)
            def _():
                idx_io(g + 3, nb, False)
                gather_io(nb, True)

            gather_io(b, False)

            @pl.when(g + 4 < NUM_CHUNKS)
            def _():
                idx_io(g + 4, b, True)

            compute(b)
            out_io(g, b, True)
        return carry

    lax.fori_loop(0, NUM_CHUNKS // NBUF, outer_body, 0)

    out_io(NUM_CHUNKS - 1, 3, False)


@jax.jit
def _embed(inputs, tok_table, pos_table):
    mesh = plsc.VectorSubcoreMesh(core_axis_name="c", subcore_axis_name="s")
    fn = pl.kernel(
        _sc_kernel_body,
        out_type=jax.ShapeDtypeStruct((BATCH, SEQ_LEN, 2 * EMBED_DIM),
                                      jnp.float32),
        mesh=mesh,
        scratch_types=(
            [pltpu.VMEM((SEQ_PER_CHUNK, SEQ_LEN), jnp.int32)] * NBUF
            + [pltpu.VMEM((ROWS_PER_CHUNK, EMBED_DIM), jnp.float32)] * NBUF
            + [pltpu.VMEM((SEQ_LEN, EMBED_DIM), jnp.float32)]
            + [pltpu.SemaphoreType.DMA] * (3 * NBUF)
        ),
        compiler_params=pltpu.CompilerParams(use_tc_tiling_on_sc=False),
    )
    return fn(inputs, tok_table, pos_table)


def kernel(inputs, tok_table, pos_table):
    return _embed(inputs, tok_table, pos_table)[:, :, :EMBED_DIM]
